# baseline (speedup 1.0000x reference)
import functools

import jax
import jax.numpy as jnp
from jax import lax
from jax.experimental import pallas as pl
from jax.experimental.pallas import tpu as pltpu
from jax.experimental.pallas import tpu_sc as plsc

_BATCH = 16384
_EMB = 64
_HID = 32

_NC = 2
_NS = 16
_NW = _NC * _NS
_BPW = _BATCH // _NW
_CHUNK = _BPW // 2


@functools.cache
def _make_sc_gather():
    @functools.partial(
        pl.kernel,
        mesh=plsc.VectorSubcoreMesh(
            core_axis_name="c", subcore_axis_name="s",
            num_cores=_NC, num_subcores=_NS,
        ),
        out_type=jax.ShapeDtypeStruct((_BATCH, 2 * _EMB), jnp.float32),
        scratch_types=[
            pltpu.VMEM((_BPW,), jnp.int32),
            pltpu.VMEM((_CHUNK, 2 * _EMB), jnp.float32),
            pltpu.VMEM((_BPW,), jnp.int32),
            pltpu.VMEM((_CHUNK, 2 * _EMB), jnp.float32),
            pltpu.SemaphoreType.DMA,
            pltpu.SemaphoreType.DMA,
        ],
        compiler_params=pltpu.CompilerParams(use_tc_tiling_on_sc=False),
    )
    def sc_gather(user_hbm, quiz_hbm, xt_hbm, x_out,
                  uidx_v, urows_v, qidx_v, qrows_v, sem_u, sem_q):
        wid = lax.axis_index("s") * _NC + lax.axis_index("c")
        base = wid * _BPW
        pltpu.sync_copy(user_hbm.at[pl.ds(base, _BPW)], uidx_v)
        pltpu.sync_copy(quiz_hbm.at[pl.ds(base, _BPW)], qidx_v)
        for k in range(_BPW // _CHUNK):
            off = k * _CHUNK
            cu = pltpu.async_copy(
                xt_hbm.at[uidx_v.at[pl.ds(off, _CHUNK)]], urows_v, sem_u)
            cq = pltpu.async_copy(
                xt_hbm.at[qidx_v.at[pl.ds(off, _CHUNK)]], qrows_v, sem_q)
            cu.wait()
            pltpu.sync_copy(urows_v.at[:, pl.ds(0, _EMB)],
                            x_out.at[pl.ds(base + off, _CHUNK), pl.ds(0, _EMB)])
            cq.wait()
            pltpu.sync_copy(qrows_v.at[:, pl.ds(_EMB, _EMB)],
                            x_out.at[pl.ds(base + off, _CHUNK),
                                     pl.ds(_EMB, _EMB)])

    return sc_gather


def _merge_body(u_ref, q_ref, out_ref):
    out_ref[...] = jnp.concatenate(
        [u_ref[...].T, q_ref[...].T], axis=1)


_MERGE_BLK = 4096


def _merge(uT, qT):
    n = uT.shape[1]
    grid = (pl.cdiv(n, _MERGE_BLK),)
    return pl.pallas_call(
        _merge_body,
        grid=grid,
        in_specs=[
            pl.BlockSpec((_EMB, _MERGE_BLK), lambda i: (0, i)),
            pl.BlockSpec((_EMB, _MERGE_BLK), lambda i: (0, i)),
        ],
        out_specs=pl.BlockSpec((_MERGE_BLK, 2 * _EMB), lambda i: (i, 0)),
        out_shape=jax.ShapeDtypeStruct((n, 2 * _EMB), jnp.float32),
    )(uT, qT)


def _mlp_body(x_ref, t_ref, w1_ref, w1t_ref, b1_ref, w2_ref, b2_ref, out_ref):
    h = (
        jnp.dot(x_ref[...], w1_ref[...], preferred_element_type=jnp.float32)
        + t_ref[...] * w1t_ref[...]
        + b1_ref[...]
    )
    h = jnp.maximum(h, 0.0)
    o = jnp.dot(h, w2_ref[...], preferred_element_type=jnp.float32) + b2_ref[...]
    out_ref[...] = 1.0 / (1.0 + jnp.exp(-o))


_MLP_BLK = 2048


def _mlp(x, time, W1x, W1t, b1, W2, b2):
    grid = (_BATCH // _MLP_BLK,)
    full = lambda shape: pl.BlockSpec(shape, lambda i: (0, 0))
    return pl.pallas_call(
        _mlp_body,
        grid=grid,
        in_specs=[
            pl.BlockSpec((_MLP_BLK, 2 * _EMB), lambda i: (i, 0)),
            pl.BlockSpec((_MLP_BLK, 1), lambda i: (i, 0)),
            full((2 * _EMB, _HID)),
            full((1, _HID)),
            full((1, _HID)),
            full((_HID, 1)),
            full((1, 1)),
        ],
        out_specs=pl.BlockSpec((_MLP_BLK, 1), lambda i: (i, 0)),
        out_shape=jax.ShapeDtypeStruct((_BATCH, 1), jnp.float32),
    )(x, time, W1x, W1t, b1, W2, b2)


def kernel(user, quiz, time, user_table, quiz_table, W1, b1, W2, b2):
    xt = _merge(user_table.T, quiz_table.T)
    x = _make_sc_gather()(user, quiz, xt)
    W1x = W1[:2 * _EMB]
    W1t = W1[2 * _EMB:]
    out = _mlp(x, time, W1x, W1t, b1.reshape(1, _HID), W2, b2.reshape(1, 1))
    return out[:, 0]


# speedup vs baseline: 1.1222x; 1.1222x over previous
import functools

import jax
import jax.numpy as jnp
from jax import lax
from jax.experimental import pallas as pl
from jax.experimental.pallas import tpu as pltpu
from jax.experimental.pallas import tpu_sc as plsc

_BATCH = 16384
_EMB = 64
_HID = 32

_NC = 2
_NS = 16
_NW = _NC * _NS
_BPW = _BATCH // _NW
_CHUNK = _BPW // 2


@functools.cache
def _make_sc_gather():
    @functools.partial(
        pl.kernel,
        mesh=plsc.VectorSubcoreMesh(
            core_axis_name="c", subcore_axis_name="s",
            num_cores=_NC, num_subcores=_NS,
        ),
        out_type=jax.ShapeDtypeStruct((_BATCH, 2 * _EMB), jnp.float32),
        scratch_types=[
            pltpu.VMEM((_BPW,), jnp.int32),
            pltpu.VMEM((_CHUNK, 2 * _EMB), jnp.float32),
            pltpu.VMEM((_BPW,), jnp.int32),
            pltpu.VMEM((_CHUNK, 2 * _EMB), jnp.float32),
            pltpu.SemaphoreType.DMA,
            pltpu.SemaphoreType.DMA,
        ],
        compiler_params=pltpu.CompilerParams(use_tc_tiling_on_sc=False),
    )
    def sc_gather(user_hbm, quiz_hbm, xt_hbm, x_out,
                  uidx_v, urows_v, qidx_v, qrows_v, sem_u, sem_q):
        wid = lax.axis_index("s") * _NC + lax.axis_index("c")
        base = wid * _BPW
        pltpu.sync_copy(user_hbm.at[pl.ds(base, _BPW)], uidx_v)
        pltpu.sync_copy(quiz_hbm.at[pl.ds(base, _BPW)], qidx_v)
        for k in range(_BPW // _CHUNK):
            off = k * _CHUNK
            cu = pltpu.async_copy(
                xt_hbm.at[uidx_v.at[pl.ds(off, _CHUNK)]], urows_v, sem_u)
            cq = pltpu.async_copy(
                xt_hbm.at[qidx_v.at[pl.ds(off, _CHUNK)]], qrows_v, sem_q)
            cu.wait()
            pltpu.sync_copy(urows_v.at[:, pl.ds(0, _EMB)],
                            x_out.at[pl.ds(base + off, _CHUNK), pl.ds(0, _EMB)])
            cq.wait()
            pltpu.sync_copy(qrows_v.at[:, pl.ds(_EMB, _EMB)],
                            x_out.at[pl.ds(base + off, _CHUNK),
                                     pl.ds(_EMB, _EMB)])

    return sc_gather


def _merge_body(u_ref, q_ref, out_ref):
    out_ref[...] = jnp.concatenate(
        [u_ref[...].T, q_ref[...].T], axis=1)


_MERGE_BLK = 8192


def _merge(uT, qT):
    n = uT.shape[1]
    grid = (pl.cdiv(n, _MERGE_BLK),)
    return pl.pallas_call(
        _merge_body,
        grid=grid,
        in_specs=[
            pl.BlockSpec((_EMB, _MERGE_BLK), lambda i: (0, i)),
            pl.BlockSpec((_EMB, _MERGE_BLK), lambda i: (0, i)),
        ],
        out_specs=pl.BlockSpec((_MERGE_BLK, 2 * _EMB), lambda i: (i, 0)),
        out_shape=jax.ShapeDtypeStruct((n, 2 * _EMB), jnp.float32),
    )(uT, qT)


def _mlp_body(x_ref, t_ref, w1_ref, w1t_ref, b1_ref, w2_ref, b2_ref, out_ref):
    h = (
        jnp.dot(x_ref[...], w1_ref[...], preferred_element_type=jnp.float32)
        + t_ref[...] * w1t_ref[...]
        + b1_ref[...]
    )
    h = jnp.maximum(h, 0.0)
    o = jnp.dot(h, w2_ref[...], preferred_element_type=jnp.float32) + b2_ref[...]
    s = 1.0 / (1.0 + jnp.exp(-o))
    out_ref[...] = s[:, 0].reshape(_MLP_BLK // 128, 128)


_MLP_BLK = 2048


def _mlp(x, time, W1x, W1t, b1, W2, b2):
    grid = (_BATCH // _MLP_BLK,)
    full = lambda shape: pl.BlockSpec(shape, lambda i: (0, 0))
    return pl.pallas_call(
        _mlp_body,
        grid=grid,
        in_specs=[
            pl.BlockSpec((_MLP_BLK, 2 * _EMB), lambda i: (i, 0)),
            pl.BlockSpec((_MLP_BLK, 1), lambda i: (i, 0)),
            full((2 * _EMB, _HID)),
            full((1, _HID)),
            full((1, _HID)),
            full((_HID, 1)),
            full((1, 1)),
        ],
        out_specs=pl.BlockSpec((_MLP_BLK // 128, 128), lambda i: (i, 0)),
        out_shape=jax.ShapeDtypeStruct((_BATCH // 128, 128), jnp.float32),
    )(x, time, W1x, W1t, b1, W2, b2)


def kernel(user, quiz, time, user_table, quiz_table, W1, b1, W2, b2):
    xt = _merge(user_table.T, quiz_table.T)
    x = _make_sc_gather()(user, quiz, xt)
    W1x = W1[:2 * _EMB]
    W1t = W1[2 * _EMB:]
    out = _mlp(x, time, W1x, W1t, b1.reshape(1, _HID), W2, b2.reshape(1, 1))
    return out.reshape(_BATCH)
